# baseline (speedup 1.0000x reference)
import functools

import jax
import jax.numpy as jnp
from jax import lax
from jax.experimental import pallas as pl
from jax.experimental.pallas import tpu as pltpu
from jax.experimental.pallas import tpu_sc as plsc

AA_REP = 20
MAX_PEP = 15
FILTER = 3
N_POCKETS = 34
OUT_L = AA_REP + FILTER - 1
X_COLS = 1 + MAX_PEP * AA_REP + N_POCKETS
O_COLS = N_POCKETS * OUT_L
POCKET_OFF = 1 + MAX_PEP * AA_REP

_P2J = {0: [0], 1: [1, 2], 2: [0, 1], 3: [2], 4: [1], 6: [2, 3], 7: [3],
        10: [4], 12: [5], 14: [6, 7], 15: [7], 17: [8], 18: [5, 6], 19: [7],
        21: [8], 22: [7, 8], 24: [8], 25: [6], 27: [4], 28: [3], 30: [2],
        31: [1], 33: [0]}

NW = 32
BLK = 16
CHUNK = 32


def _splat(v):
    return jnp.full((BLK,), v, dtype=jnp.int32)


def _tec_kernel(x_hbm, w_hbm, out_hbm, x_v, o_v, w_v):
    wid = lax.axis_index("s") * 2 + lax.axis_index("c")
    per_w = out_hbm.shape[0] // (NW * O_COLS)
    base_w = wid * per_w

    pltpu.sync_copy(w_hbm, w_v)

    zero = jnp.zeros((BLK,), jnp.float32)
    zero_cols = [p * OUT_L + l for p in range(N_POCKETS) if p not in _P2J
                 for l in range(OUT_L)]
    for blk in range(CHUNK // BLK):
        rowo = (jnp.arange(BLK, dtype=jnp.int32) + blk * BLK) * O_COLS
        for c in zero_cols:
            plsc.store_scatter(o_v, [rowo + c], zero)

    def block_body(blk, carry):
        row = jnp.arange(BLK, dtype=jnp.int32) + blk * BLK
        rowx = row * X_COLS
        rowo = row * O_COLS
        for p, js in _P2J.items():
            encf = plsc.load_gather(x_v, [rowx + (POCKET_OFF + p)])
            e3 = encf.astype(jnp.int32) * FILTER
            f = [plsc.load_gather(w_v, [e3 + t]) for t in range(FILTER)]
            a = []
            for k in range(AA_REP):
                v = plsc.load_gather(x_v, [rowx + (1 + js[0] * AA_REP + k)])
                for j in js[1:]:
                    v = v + plsc.load_gather(x_v, [rowx + (1 + j * AA_REP + k)])
                a.append(v)
            for l in range(OUT_L):
                acc = None
                for t in range(FILTER):
                    k = l - t
                    if 0 <= k < AA_REP:
                        term = f[t] * a[k]
                        acc = term if acc is None else acc + term
                plsc.store_scatter(o_v, [rowo + (p * OUT_L + l)], acc)
        return carry

    def chunk_body(ci, carry):
        base = base_w + ci * CHUNK
        pltpu.sync_copy(x_hbm.at[pl.ds(base * X_COLS, CHUNK * X_COLS)], x_v)
        lax.fori_loop(0, CHUNK // BLK, block_body, 0)
        pltpu.sync_copy(o_v, out_hbm.at[pl.ds(base * O_COLS, CHUNK * O_COLS)])
        return carry

    lax.fori_loop(0, per_w // CHUNK, chunk_body, 0)


@jax.jit
def kernel(x, W):
    B = x.shape[0]
    w_pad = jnp.zeros((64,), jnp.float32).at[:AA_REP * FILTER].set(W.reshape(-1))

    mesh = plsc.VectorSubcoreMesh(core_axis_name="c", subcore_axis_name="s")
    run = functools.partial(
        pl.kernel,
        mesh=mesh,
        compiler_params=pltpu.CompilerParams(use_tc_tiling_on_sc=False,
                                              needs_layout_passes=False),
        out_type=jax.ShapeDtypeStruct((B * O_COLS,), jnp.float32),
        scratch_types=[
            pltpu.VMEM((CHUNK * X_COLS,), jnp.float32),
            pltpu.VMEM((CHUNK * O_COLS,), jnp.float32),
            pltpu.VMEM((64,), jnp.float32),
        ],
    )(_tec_kernel)
    out = run(x.reshape(-1), w_pad)
    return out.reshape(B, N_POCKETS, OUT_L)


# speedup vs baseline: 2.0638x; 2.0638x over previous
import functools

import jax
import jax.numpy as jnp
from jax import lax
from jax.experimental import pallas as pl
from jax.experimental.pallas import tpu as pltpu
from jax.experimental.pallas import tpu_sc as plsc

AA_REP = 20
MAX_PEP = 15
FILTER = 3
N_POCKETS = 34
OUT_L = AA_REP + FILTER - 1
X_COLS = 1 + MAX_PEP * AA_REP + N_POCKETS
O_COLS = N_POCKETS * OUT_L
POCKET_OFF = 1 + MAX_PEP * AA_REP

_P2J = {0: [0], 1: [1, 2], 2: [0, 1], 3: [2], 4: [1], 6: [2, 3], 7: [3],
        10: [4], 12: [5], 14: [6, 7], 15: [7], 17: [8], 18: [5, 6], 19: [7],
        21: [8], 22: [7, 8], 24: [8], 25: [6], 27: [4], 28: [3], 30: [2],
        31: [1], 33: [0]}

NW = 32
BLK = 16
CHUNK = 32


def _splat(v):
    return jnp.full((BLK,), v, dtype=jnp.int32)


def _zero_cols(o_v):
    zero = jnp.zeros((BLK,), jnp.float32)
    cols = [p * OUT_L + l for p in range(N_POCKETS) if p not in _P2J
            for l in range(OUT_L)]
    for blk in range(CHUNK // BLK):
        row = jnp.arange(BLK, dtype=jnp.int32) + blk * BLK
        for c in cols:
            plsc.store_scatter(o_v, [row, _splat(c)], zero)


def _compute_chunk(x_v, o_v, w_v):
    def block_body(blk, carry):
        row = jnp.arange(BLK, dtype=jnp.int32) + blk * BLK
        for p, js in _P2J.items():
            encf = plsc.load_gather(x_v, [row, _splat(POCKET_OFF + p)])
            e3 = encf.astype(jnp.int32) * FILTER
            f = [plsc.load_gather(w_v, [e3 + t]) for t in range(FILTER)]
            a = []
            for k in range(AA_REP):
                v = plsc.load_gather(x_v, [row, _splat(1 + js[0] * AA_REP + k)])
                for j in js[1:]:
                    v = v + plsc.load_gather(x_v, [row, _splat(1 + j * AA_REP + k)])
                a.append(v)
            for l in range(OUT_L):
                acc = None
                for t in range(FILTER):
                    k = l - t
                    if 0 <= k < AA_REP:
                        term = f[t] * a[k]
                        acc = term if acc is None else acc + term
                plsc.store_scatter(o_v, [row, _splat(p * OUT_L + l)], acc)
        return carry

    lax.fori_loop(0, CHUNK // BLK, block_body, 0)


def _tec_kernel(x_hbm, w_hbm, out_hbm, x_v0, x_v1, o_v0, o_v1, w_v,
                si0, si1, so0, so1):
    wid = lax.axis_index("s") * 2 + lax.axis_index("c")
    per_w = out_hbm.shape[0] // NW
    base_w = wid * per_w
    n_pairs = per_w // (2 * CHUNK)

    pltpu.sync_copy(w_hbm, w_v)
    _zero_cols(o_v0)
    _zero_cols(o_v1)

    def in_cp(base, buf, sem):
        return pltpu.make_async_copy(x_hbm.at[pl.ds(base, CHUNK)], buf, sem)

    def out_cp(base, buf, sem):
        return pltpu.make_async_copy(buf, out_hbm.at[pl.ds(base, CHUNK)], sem)

    in_cp(base_w, x_v0, si0).start()

    def pair_body(g, carry):
        c0 = base_w + 2 * g * CHUNK
        c1 = c0 + CHUNK
        in_cp(c1, x_v1, si1).start()
        in_cp(c0, x_v0, si0).wait()

        @pl.when(g > 0)
        def _():
            out_cp(c0, o_v0, so0).wait()
        _compute_chunk(x_v0, o_v0, w_v)
        out_cp(c0, o_v0, so0).start()

        @pl.when(g + 1 < n_pairs)
        def _():
            in_cp(c1 + CHUNK, x_v0, si0).start()
        in_cp(c1, x_v1, si1).wait()

        @pl.when(g > 0)
        def _():
            out_cp(c1, o_v1, so1).wait()
        _compute_chunk(x_v1, o_v1, w_v)
        out_cp(c1, o_v1, so1).start()
        return carry

    lax.fori_loop(0, n_pairs, pair_body, 0)
    out_cp(base_w, o_v0, so0).wait()
    out_cp(base_w, o_v1, so1).wait()


@jax.jit
def kernel(x, W):
    B = x.shape[0]
    w_pad = jnp.zeros((64,), jnp.float32).at[:AA_REP * FILTER].set(W.reshape(-1))

    mesh = plsc.VectorSubcoreMesh(core_axis_name="c", subcore_axis_name="s")
    run = functools.partial(
        pl.kernel,
        mesh=mesh,
        compiler_params=pltpu.CompilerParams(use_tc_tiling_on_sc=False,
                                             needs_layout_passes=False),
        out_type=jax.ShapeDtypeStruct((B, O_COLS), jnp.float32),
        scratch_types=[
            pltpu.VMEM((CHUNK, X_COLS), jnp.float32),
            pltpu.VMEM((CHUNK, X_COLS), jnp.float32),
            pltpu.VMEM((CHUNK, O_COLS), jnp.float32),
            pltpu.VMEM((CHUNK, O_COLS), jnp.float32),
            pltpu.VMEM((64,), jnp.float32),
            pltpu.SemaphoreType.DMA,
            pltpu.SemaphoreType.DMA,
            pltpu.SemaphoreType.DMA,
            pltpu.SemaphoreType.DMA,
        ],
    )(_tec_kernel)
    out = run(x, w_pad)
    return out.reshape(B, N_POCKETS, OUT_L)


# speedup vs baseline: 3.4005x; 1.6477x over previous
import functools

import jax
import jax.numpy as jnp
from jax import lax
from jax.experimental import pallas as pl
from jax.experimental.pallas import tpu as pltpu
from jax.experimental.pallas import tpu_sc as plsc

AA_REP = 20
MAX_PEP = 15
FILTER = 3
N_POCKETS = 34
OUT_L = AA_REP + FILTER - 1
X_COLS = 1 + MAX_PEP * AA_REP + N_POCKETS
O_COLS = N_POCKETS * OUT_L
POCKET_OFF = 1 + MAX_PEP * AA_REP

_P2J = {0: [0], 1: [1, 2], 2: [0, 1], 3: [2], 4: [1], 6: [2, 3], 7: [3],
        10: [4], 12: [5], 14: [6, 7], 15: [7], 17: [8], 18: [5, 6], 19: [7],
        21: [8], 22: [7, 8], 24: [8], 25: [6], 27: [4], 28: [3], 30: [2],
        31: [1], 33: [0]}

NW = 32
BLK = 16
CHUNK = 32


def _zero_rows(o_v):
    zero = jnp.zeros((BLK,), jnp.float32)
    rows = [p * OUT_L + l for p in range(N_POCKETS) if p not in _P2J
            for l in range(OUT_L)]
    for r in rows:
        for blk in range(CHUNK // BLK):
            o_v[r, pl.ds(blk * BLK, BLK)] = zero


def _compute_chunk(x_v, o_v, w_v):
    def block_body(blk, carry):
        s0 = blk * BLK
        for p, js in _P2J.items():
            encf = x_v[POCKET_OFF + p, pl.ds(s0, BLK)]
            e3 = encf.astype(jnp.int32) * FILTER
            f = [plsc.load_gather(w_v, [e3 + t]) for t in range(FILTER)]
            a = []
            for k in range(AA_REP):
                v = x_v[1 + js[0] * AA_REP + k, pl.ds(s0, BLK)]
                for j in js[1:]:
                    v = v + x_v[1 + j * AA_REP + k, pl.ds(s0, BLK)]
                a.append(v)
            for l in range(OUT_L):
                acc = None
                for t in range(FILTER):
                    k = l - t
                    if 0 <= k < AA_REP:
                        term = f[t] * a[k]
                        acc = term if acc is None else acc + term
                o_v[p * OUT_L + l, pl.ds(s0, BLK)] = acc
        return carry

    lax.fori_loop(0, CHUNK // BLK, block_body, 0)


def _tec_kernel(x_hbm, w_hbm, out_hbm, x_v0, x_v1, o_v0, o_v1, w_v,
                si0, si1, so0, so1):
    wid = lax.axis_index("s") * 2 + lax.axis_index("c")
    per_w = out_hbm.shape[1] // NW
    base_w = wid * per_w
    n_pairs = per_w // (2 * CHUNK)

    pltpu.sync_copy(w_hbm, w_v)
    _zero_rows(o_v0)
    _zero_rows(o_v1)

    def in_cp(base, buf, sem):
        return pltpu.make_async_copy(x_hbm.at[:, pl.ds(base, CHUNK)], buf, sem)

    def out_cp(base, buf, sem):
        return pltpu.make_async_copy(buf, out_hbm.at[:, pl.ds(base, CHUNK)], sem)

    in_cp(base_w, x_v0, si0).start()

    def pair_body(g, carry):
        c0 = base_w + 2 * g * CHUNK
        c1 = c0 + CHUNK
        in_cp(c1, x_v1, si1).start()
        in_cp(c0, x_v0, si0).wait()

        @pl.when(g > 0)
        def _():
            out_cp(c0, o_v0, so0).wait()
        _compute_chunk(x_v0, o_v0, w_v)
        out_cp(c0, o_v0, so0).start()

        @pl.when(g + 1 < n_pairs)
        def _():
            in_cp(c1 + CHUNK, x_v0, si0).start()
        in_cp(c1, x_v1, si1).wait()

        @pl.when(g > 0)
        def _():
            out_cp(c1, o_v1, so1).wait()
        _compute_chunk(x_v1, o_v1, w_v)
        out_cp(c1, o_v1, so1).start()
        return carry

    lax.fori_loop(0, n_pairs, pair_body, 0)
    out_cp(base_w, o_v0, so0).wait()
    out_cp(base_w, o_v1, so1).wait()


@jax.jit
def kernel(x, W):
    B = x.shape[0]
    w_pad = jnp.zeros((64,), jnp.float32).at[:AA_REP * FILTER].set(W.reshape(-1))

    mesh = plsc.VectorSubcoreMesh(core_axis_name="c", subcore_axis_name="s")
    run = functools.partial(
        pl.kernel,
        mesh=mesh,
        compiler_params=pltpu.CompilerParams(use_tc_tiling_on_sc=False,
                                             needs_layout_passes=False),
        out_type=jax.ShapeDtypeStruct((O_COLS, B), jnp.float32),
        scratch_types=[
            pltpu.VMEM((X_COLS, CHUNK), jnp.float32),
            pltpu.VMEM((X_COLS, CHUNK), jnp.float32),
            pltpu.VMEM((O_COLS, CHUNK), jnp.float32),
            pltpu.VMEM((O_COLS, CHUNK), jnp.float32),
            pltpu.VMEM((64,), jnp.float32),
            pltpu.SemaphoreType.DMA,
            pltpu.SemaphoreType.DMA,
            pltpu.SemaphoreType.DMA,
            pltpu.SemaphoreType.DMA,
        ],
    )(_tec_kernel)
    out_t = run(x.T, w_pad)
    return out_t.T.reshape(B, N_POCKETS, OUT_L)


# speedup vs baseline: 6.2607x; 1.8411x over previous
import functools

import jax
import jax.numpy as jnp
from jax import lax
from jax.experimental import pallas as pl
from jax.experimental.layout import Format, Layout
from jax.experimental.pallas import tpu as pltpu
from jax.experimental.pallas import tpu_sc as plsc

AA_REP = 20
MAX_PEP = 15
FILTER = 3
N_POCKETS = 34
OUT_L = AA_REP + FILTER - 1
X_COLS = 1 + MAX_PEP * AA_REP + N_POCKETS
O_COLS = N_POCKETS * OUT_L
OUT_PAD = OUT_L
POCKET_OFF = 1 + MAX_PEP * AA_REP

_P2J = {0: [0], 1: [1, 2], 2: [0, 1], 3: [2], 4: [1], 6: [2, 3], 7: [3],
        10: [4], 12: [5], 14: [6, 7], 15: [7], 17: [8], 18: [5, 6], 19: [7],
        21: [8], 22: [7, 8], 24: [8], 25: [6], 27: [4], 28: [3], 30: [2],
        31: [1], 33: [0]}

NW = 32
BLK = 16
CHUNK = 32


def _zero_rows(o_v):
    zero = jnp.zeros((BLK,), jnp.float32)
    for p in range(N_POCKETS):
        if p in _P2J:
            continue
        for l in range(OUT_L):
            for blk in range(CHUNK // BLK):
                o_v[p, l, pl.ds(blk * BLK, BLK)] = zero


def _compute_chunk(x_v, o_v, w0, w1, w2):
    pockets = list(_P2J.items())

    def block_body(blk, carry):
        s0 = blk * BLK

        def enc(p):
            return x_v[POCKET_OFF + p, pl.ds(s0, BLK)].astype(jnp.int32)

        def taps(e):
            return (plsc.load_gather(w0, [e]), plsc.load_gather(w1, [e]),
                    plsc.load_gather(w2, [e]))

        def ldpos(j):
            return [x_v[1 + j * AA_REP + k, pl.ds(s0, BLK)]
                    for k in range(AA_REP)]

        def conv(p, a, f):
            for l in range(OUT_L):
                acc = None
                for t in range(FILTER):
                    k = l - t
                    if 0 <= k < AA_REP:
                        term = f[t] * a[k]
                        acc = term if acc is None else acc + term
                o_v[p, l, pl.ds(s0, BLK)] = acc

        sched = []
        for p, js in pockets:
            sched.append((max(js), p, js))
        sched.sort()
        f = taps(enc(sched[0][1]))
        prev = None
        cur = ldpos(0)
        si = 0
        for j in range(MAX_PEP):
            if si >= len(sched):
                break
            if sched[si][0] > j:
                prev, cur = cur, ldpos(j + 1)
                continue
            while si < len(sched) and sched[si][0] == j:
                _, p, js = sched[si]
                f_next = (taps(enc(sched[si + 1][1]))
                          if si + 1 < len(sched) else None)
                if len(js) == 1:
                    a = cur
                else:
                    a = [prev[k] + cur[k] for k in range(AA_REP)]
                conv(p, a, f)
                f = f_next
                si += 1
            if si < len(sched):
                prev, cur = cur, ldpos(j + 1)
        return carry

    lax.fori_loop(0, CHUNK // BLK, block_body, 0)


def _tec_kernel(x_hbm, w_hbm, out_hbm, x_v0, x_v1, o_v0, o_v1, w_v,
                si0, si1, so0, so1):
    w0 = w_v.at[0]
    w1 = w_v.at[1]
    w2 = w_v.at[2]
    wid = lax.axis_index("s") * 2 + lax.axis_index("c")
    per_w = out_hbm.shape[2] // NW
    base_w = wid * per_w
    n_pairs = per_w // (2 * CHUNK)

    pltpu.sync_copy(w_hbm, w_v)
    _zero_rows(o_v0)
    _zero_rows(o_v1)

    def in_cp(base, buf, sem):
        return pltpu.make_async_copy(x_hbm.at[:, pl.ds(base, CHUNK)], buf, sem)

    def out_cp(base, buf, sem):
        return pltpu.make_async_copy(buf, out_hbm.at[:, :, pl.ds(base, CHUNK)],
                                     sem)

    in_cp(base_w, x_v0, si0).start()

    def pair_body(g, carry):
        c0 = base_w + 2 * g * CHUNK
        c1 = c0 + CHUNK
        in_cp(c1, x_v1, si1).start()
        in_cp(c0, x_v0, si0).wait()

        @pl.when(g > 0)
        def _():
            out_cp(c0, o_v0, so0).wait()
        _compute_chunk(x_v0, o_v0, w0, w1, w2)
        out_cp(c0, o_v0, so0).start()

        @pl.when(g + 1 < n_pairs)
        def _():
            in_cp(c1 + CHUNK, x_v0, si0).start()
        in_cp(c1, x_v1, si1).wait()

        @pl.when(g > 0)
        def _():
            out_cp(c1, o_v1, so1).wait()
        _compute_chunk(x_v1, o_v1, w0, w1, w2)
        out_cp(c1, o_v1, so1).start()
        return carry

    lax.fori_loop(0, n_pairs, pair_body, 0)
    out_cp(base_w, o_v0, so0).wait()
    out_cp(base_w, o_v1, so1).wait()


def _kernel_impl(x, W):
    B = x.shape[0]
    w_pad = jnp.zeros((FILTER, 32), jnp.float32).at[:, :AA_REP].set(W.T)

    mesh = plsc.VectorSubcoreMesh(core_axis_name="c", subcore_axis_name="s")
    run = functools.partial(
        pl.kernel,
        mesh=mesh,
        compiler_params=pltpu.CompilerParams(use_tc_tiling_on_sc=False,
                                             needs_layout_passes=False),
        out_type=jax.ShapeDtypeStruct((N_POCKETS, OUT_PAD, B), jnp.float32),
        scratch_types=[
            pltpu.VMEM((X_COLS, CHUNK), jnp.float32),
            pltpu.VMEM((X_COLS, CHUNK), jnp.float32),
            pltpu.VMEM((N_POCKETS, OUT_PAD, CHUNK), jnp.float32),
            pltpu.VMEM((N_POCKETS, OUT_PAD, CHUNK), jnp.float32),
            pltpu.VMEM((FILTER, 32), jnp.float32),
            pltpu.SemaphoreType.DMA,
            pltpu.SemaphoreType.DMA,
            pltpu.SemaphoreType.DMA,
            pltpu.SemaphoreType.DMA,
        ],
    )(_tec_kernel)
    out_t = run(x.T, w_pad)
    return out_t.transpose(2, 0, 1)


kernel = jax.jit(_kernel_impl)


# speedup vs baseline: 6.2678x; 1.0011x over previous
import functools

import jax
import jax.numpy as jnp
from jax import lax
from jax.experimental import pallas as pl
from jax.experimental.pallas import tpu as pltpu
from jax.experimental.pallas import tpu_sc as plsc

AA_REP = 20
MAX_PEP = 15
FILTER = 3
N_POCKETS = 34
OUT_L = AA_REP + FILTER - 1
X_COLS = 1 + MAX_PEP * AA_REP + N_POCKETS
O_COLS = N_POCKETS * OUT_L
OUT_PAD = OUT_L
POCKET_OFF = 1 + MAX_PEP * AA_REP

_P2J = {0: [0], 1: [1, 2], 2: [0, 1], 3: [2], 4: [1], 6: [2, 3], 7: [3],
        10: [4], 12: [5], 14: [6, 7], 15: [7], 17: [8], 18: [5, 6], 19: [7],
        21: [8], 22: [7, 8], 24: [8], 25: [6], 27: [4], 28: [3], 30: [2],
        31: [1], 33: [0]}

NW = 32
BLK = 16
CHUNK = 32


def _zero_rows(o_v):
    zero = jnp.zeros((BLK,), jnp.float32)
    for p in range(N_POCKETS):
        if p in _P2J:
            continue
        for l in range(OUT_L):
            for blk in range(CHUNK // BLK):
                o_v[p, l, pl.ds(blk * BLK, BLK)] = zero


def _compute_chunk(x_v, o_v, w0, w1, w2):
    pockets = list(_P2J.items())

    def block_body(blk, carry):
        s0 = blk * BLK

        def enc(p):
            return x_v[POCKET_OFF + p, pl.ds(s0, BLK)].astype(jnp.int32)

        def taps(e):
            return (plsc.load_gather(w0, [e]), plsc.load_gather(w1, [e]),
                    plsc.load_gather(w2, [e]))

        def ldpos(j):
            return [x_v[1 + j * AA_REP + k, pl.ds(s0, BLK)]
                    for k in range(AA_REP)]

        def conv(p, a, f):
            for l in range(OUT_L):
                acc = None
                for t in range(FILTER):
                    k = l - t
                    if 0 <= k < AA_REP:
                        term = f[t] * a[k]
                        acc = term if acc is None else acc + term
                o_v[p, l, pl.ds(s0, BLK)] = acc

        sched = []
        for p, js in pockets:
            sched.append((max(js), p, js))
        sched.sort()
        f = taps(enc(sched[0][1]))
        prev = None
        cur = ldpos(0)
        si = 0
        for j in range(MAX_PEP):
            if si >= len(sched):
                break
            if sched[si][0] > j:
                prev, cur = cur, ldpos(j + 1)
                continue
            while si < len(sched) and sched[si][0] == j:
                _, p, js = sched[si]
                f_next = (taps(enc(sched[si + 1][1]))
                          if si + 1 < len(sched) else None)
                if len(js) == 1:
                    a = cur
                else:
                    a = [prev[k] + cur[k] for k in range(AA_REP)]
                conv(p, a, f)
                f = f_next
                si += 1
            if si < len(sched):
                prev, cur = cur, ldpos(j + 1)
        return carry

    lax.fori_loop(0, CHUNK // BLK, block_body, 0)


def _tec_kernel(x_hbm, w_hbm, out_hbm, x_v0, x_v1, o_v0, o_v1, w_v,
                si0, si1, so0, so1):
    w0 = w_v.at[0]
    w1 = w_v.at[1]
    w2 = w_v.at[2]
    wid = lax.axis_index("s") * 2 + lax.axis_index("c")
    per_w = out_hbm.shape[2] // NW
    base_w = wid * per_w
    n_pairs = per_w // (2 * CHUNK)

    pltpu.sync_copy(w_hbm, w_v)
    _zero_rows(o_v0)
    _zero_rows(o_v1)

    def in_cp(base, buf, sem):
        return pltpu.make_async_copy(x_hbm.at[:, pl.ds(base, CHUNK)], buf, sem)

    def out_cp(base, buf, sem):
        return pltpu.make_async_copy(buf, out_hbm.at[:, :, pl.ds(base, CHUNK)],
                                     sem)

    in_cp(base_w, x_v0, si0).start()

    def pair_body(g, carry):
        c0 = base_w + 2 * g * CHUNK
        c1 = c0 + CHUNK
        in_cp(c1, x_v1, si1).start()
        in_cp(c0, x_v0, si0).wait()

        @pl.when(g > 0)
        def _():
            out_cp(c0, o_v0, so0).wait()
        _compute_chunk(x_v0, o_v0, w0, w1, w2)
        out_cp(c0, o_v0, so0).start()

        @pl.when(g + 1 < n_pairs)
        def _():
            in_cp(c1 + CHUNK, x_v0, si0).start()
        in_cp(c1, x_v1, si1).wait()

        @pl.when(g > 0)
        def _():
            out_cp(c1, o_v1, so1).wait()
        _compute_chunk(x_v1, o_v1, w0, w1, w2)
        out_cp(c1, o_v1, so1).start()
        return carry

    lax.fori_loop(0, n_pairs, pair_body, 0)
    out_cp(base_w, o_v0, so0).wait()
    out_cp(base_w, o_v1, so1).wait()


def _kernel_impl(x, W):
    B = x.shape[0]
    w_pad = jnp.zeros((FILTER, 32), jnp.float32).at[:, :AA_REP].set(W.T)

    mesh = plsc.VectorSubcoreMesh(core_axis_name="c", subcore_axis_name="s")
    run = functools.partial(
        pl.kernel,
        mesh=mesh,
        compiler_params=pltpu.CompilerParams(use_tc_tiling_on_sc=False,
                                             needs_layout_passes=False),
        out_type=jax.ShapeDtypeStruct((N_POCKETS, OUT_PAD, B), jnp.float32),
        scratch_types=[
            pltpu.VMEM((X_COLS, CHUNK), jnp.float32),
            pltpu.VMEM((X_COLS, CHUNK), jnp.float32),
            pltpu.VMEM((N_POCKETS, OUT_PAD, CHUNK), jnp.float32),
            pltpu.VMEM((N_POCKETS, OUT_PAD, CHUNK), jnp.float32),
            pltpu.VMEM((FILTER, 32), jnp.float32),
            pltpu.SemaphoreType.DMA,
            pltpu.SemaphoreType.DMA,
            pltpu.SemaphoreType.DMA,
            pltpu.SemaphoreType.DMA,
        ],
    )(_tec_kernel)
    out_t = run(x.T, w_pad)
    return out_t.transpose(2, 0, 1)


kernel = jax.jit(_kernel_impl)
